# baseline (speedup 1.0000x reference)
import functools

import jax
import jax.numpy as jnp
from jax import lax
from jax.experimental import pallas as pl
from jax.experimental.pallas import tpu as pltpu
from jax.experimental.pallas import tpu_sc as plsc

_COMMITMENT = 0.25
_EPS = 1e-12


def _wprep_kernel(w1_ref, wf_ref, wa_ref, wn_ref):
    w = w1_ref[...]
    n2 = jnp.sum(w * w, axis=1, keepdims=True)
    wn = w / jnp.maximum(jnp.sqrt(n2), _EPS)
    wsq = jnp.sum(wn * wn, axis=1, keepdims=True)
    wa_ref[...] = jnp.concatenate(
        [-2.0 * wn, wsq, jnp.ones_like(wsq)], axis=1)

    wf = wf_ref[...]
    nf2 = jnp.sum(wf * wf, axis=1, keepdims=True)
    wnf = wf / jnp.maximum(jnp.sqrt(nf2), _EPS)
    wn_ref[...] = jnp.concatenate([wnf, jnp.zeros_like(wnf)], axis=1)


def _vq_tile_kernel(x_ref, w_ref, loss_ref, idx_ref):
    wa = w_ref[...]

    x = x_ref[0]
    xn2 = jnp.sum(x * x, axis=0, keepdims=True)
    xn = x / jnp.maximum(jnp.sqrt(xn2), _EPS)
    xsq = jnp.sum(xn * xn, axis=0, keepdims=True)
    xa = jnp.concatenate([xn, jnp.ones_like(xsq), xsq], axis=0)

    dist = jnp.dot(wa, xa, preferred_element_type=jnp.float32)

    idx0 = jnp.argmin(dist, axis=0)
    minv = jnp.min(dist, axis=0)

    loss_ref[0, 0, :] = ((1.0 + _COMMITMENT) / 64.0) * minv
    idx_ref[0, 0, :] = (idx0 + 1).astype(jnp.int32)


def _sc_lookup(table_hbm, idx_hbm, out_hbm, idx_v, rows_v, sem, *,
               nc, b_per_w, chunk):
    wid = lax.axis_index("s") * nc + lax.axis_index("c")
    base = wid * b_per_w
    for ci in range(b_per_w // chunk):
        off = base + ci * chunk
        pltpu.sync_copy(idx_hbm.at[pl.ds(off, chunk)], idx_v)
        pltpu.async_copy(table_hbm.at[idx_v], rows_v, sem).wait()
        pltpu.sync_copy(rows_v, out_hbm.at[pl.ds(off, chunk)])


def kernel(inputs, W):
    B, C, nz, nt, nr = inputs.shape
    S = nz * nt * nr
    x3 = inputs.reshape(B, C, S)
    W1 = W[1:]
    K = W1.shape[0]

    wa, wn = pl.pallas_call(
        _wprep_kernel,
        out_shape=[
            jax.ShapeDtypeStruct((K, C + 2), jnp.float32),
            jax.ShapeDtypeStruct((K + 1, 2 * C), jnp.float32),
        ],
    )(W1, W)

    T = 4096

    tc_call = pl.pallas_call(
        _vq_tile_kernel,
        grid=(S // T,),
        in_specs=[
            pl.BlockSpec((1, C, T), lambda t: (0, 0, t)),
            pl.BlockSpec((K, C + 2), lambda t: (0, 0)),
        ],
        out_specs=[
            pl.BlockSpec((1, 1, T), lambda t: (0, 0, t)),
            pl.BlockSpec((1, 1, T), lambda t: (0, 0, t)),
        ],
        out_shape=[
            jax.ShapeDtypeStruct((1, 1, S), jnp.float32),
            jax.ShapeDtypeStruct((1, 1, S), jnp.int32),
        ],
        compiler_params=pltpu.CompilerParams(
            dimension_semantics=("parallel",)),
    )

    info = plsc.get_sparse_core_info()
    nc, ns = info.num_cores, info.num_subcores
    nw = nc * ns
    b_per_w = S // nw
    chunk = min(b_per_w, 512)

    body = functools.partial(_sc_lookup, nc=nc, b_per_w=b_per_w, chunk=chunk)
    sc_call = pl.kernel(
        body,
        mesh=plsc.VectorSubcoreMesh(core_axis_name="c", subcore_axis_name="s"),
        out_type=jax.ShapeDtypeStruct((S, 2 * C), jnp.float32),
        scratch_types=[
            pltpu.VMEM((chunk,), jnp.int32),
            pltpu.VMEM((chunk, 2 * C), jnp.float32),
            pltpu.SemaphoreType.DMA,
        ],
    )

    loss_parts, idx_parts, q_parts = [], [], []
    for b in range(B):
        loss_b, idx_b = tc_call(x3[b:b + 1], wa)
        qtok_b = sc_call(wn, idx_b.reshape(S))
        q_parts.append(qtok_b[:, :C].T)
        loss_parts.append(loss_b)
        idx_parts.append(idx_b)

    quantized_out = jnp.stack(q_parts).reshape(B, C, nz, nt, nr)
    vq_loss_spatial = jnp.concatenate(loss_parts).reshape(B, nz, nt, nr)
    spatial_indices = jnp.concatenate(idx_parts).reshape(B, nz, nt, nr)
    return quantized_out, vq_loss_spatial, spatial_indices


# speedup vs baseline: 1.5049x; 1.5049x over previous
import jax
import jax.numpy as jnp
from jax.experimental import pallas as pl
from jax.experimental.pallas import tpu as pltpu

_COMMITMENT = 0.25
_EPS = 1e-12


def _wprep_kernel(w_ref, wa_ref):
    w = w_ref[...]
    wn2 = jnp.sum(w * w, axis=1, keepdims=True)
    wn = w / jnp.maximum(jnp.sqrt(wn2), _EPS)
    wsq = jnp.sum(wn * wn, axis=1, keepdims=True)
    wa_ref[...] = jnp.concatenate(
        [-2.0 * wn, wsq, jnp.ones_like(wsq)], axis=1)


def _vq_tile_kernel(x_ref, w_ref, q_ref, loss_ref, idx_ref):
    wa = w_ref[...]

    x = x_ref[0]
    xn2 = jnp.sum(x * x, axis=0, keepdims=True)
    xn = x / jnp.maximum(jnp.sqrt(xn2), _EPS)
    xsq = jnp.sum(xn * xn, axis=0, keepdims=True)
    xa = jnp.concatenate([xn, jnp.ones_like(xsq), xsq], axis=0)

    dist = jnp.dot(wa, xa, precision=jax.lax.Precision.DEFAULT,
                   preferred_element_type=jnp.float32)

    idx0 = jnp.argmin(dist, axis=0)

    onehot = (jax.lax.broadcasted_iota(jnp.int32, dist.shape, 0)
              == idx0[None, :]).astype(jnp.float32)
    q = -0.5 * jax.lax.dot_general(wa[:, :64], onehot,
                                   dimension_numbers=(((0,), (0,)), ((), ())),
                                   preferred_element_type=jnp.float32)

    d = q - xn
    sq = d * d
    loss = jnp.mean(sq + _COMMITMENT * sq, axis=0)

    q_ref[0] = q
    loss_ref[0, 0, :] = loss
    idx_ref[0, 0, :] = (idx0 + 1).astype(jnp.int32)


def kernel(inputs, W):
    B, C, nz, nt, nr = inputs.shape
    S = nz * nt * nr
    x3 = inputs.reshape(B, C, S)
    W1 = W[1:]
    K = W1.shape[0]

    wa = pl.pallas_call(
        _wprep_kernel,
        out_shape=jax.ShapeDtypeStruct((K, C + 2), jnp.float32),
    )(W1)

    T = 4096
    grid = (B, S // T)

    q3, loss3, idx3 = pl.pallas_call(
        _vq_tile_kernel,
        grid=grid,
        in_specs=[
            pl.BlockSpec((1, C, T), lambda b, t: (b, 0, t)),
            pl.BlockSpec((K, C + 2), lambda b, t: (0, 0)),
        ],
        out_specs=[
            pl.BlockSpec((1, C, T), lambda b, t: (b, 0, t)),
            pl.BlockSpec((1, 1, T), lambda b, t: (b, 0, t)),
            pl.BlockSpec((1, 1, T), lambda b, t: (b, 0, t)),
        ],
        out_shape=[
            jax.ShapeDtypeStruct((B, C, S), jnp.float32),
            jax.ShapeDtypeStruct((B, 1, S), jnp.float32),
            jax.ShapeDtypeStruct((B, 1, S), jnp.int32),
        ],
        compiler_params=pltpu.CompilerParams(
            dimension_semantics=("parallel", "parallel")),
    )(x3, wa)

    quantized_out = q3.reshape(B, C, nz, nt, nr)
    vq_loss_spatial = loss3.reshape(B, nz, nt, nr)
    spatial_indices = idx3.reshape(B, nz, nt, nr)
    return quantized_out, vq_loss_spatial, spatial_indices
